# baseline (speedup 1.0000x reference)
import math

import jax
import jax.numpy as jnp
from jax import lax
from jax.experimental import pallas as pl
from jax.experimental.pallas import tpu as pltpu

D_MODEL = 768
HEADS = 12
D_KEY = D_MODEL // HEADS
E = 8
D_FF = D_MODEL * 4
S = 2048
BLK = 256
SBLK = 256
ABLK = 1024
NCH = 4
FCH = D_FF // NCH

_NT = (((1,), (1,)), ((), ()))


def _rms(x, scale, eps=1e-5):
    r = jnp.sqrt(jnp.mean(x * x, axis=-1, keepdims=True) + eps)
    return scale * (x / r)


def _qkv_kernel(x_ref, sc_ref, wq_ref, wk_ref, wv_ref, q_ref, k_ref, v_ref):
    a = _rms(x_ref[...], sc_ref[...])
    q_ref[...] = lax.dot_general(a, wq_ref[...], _NT,
                                 preferred_element_type=jnp.float32)
    k_ref[...] = lax.dot_general(a, wk_ref[...], _NT,
                                 preferred_element_type=jnp.float32)
    v_ref[...] = lax.dot_general(a, wv_ref[...], _NT,
                                 preferred_element_type=jnp.float32)


def _arr_kernel(x_ref, q_ref, k_ref, v_ref, wo_ref, sc_ref, gw_ref,
                x1_ref, f_ref, probs_ref, e0_ref, e1_ref, g0_ref, g1_ref):
    outs = []
    for h in range(HEADS):
        lo = h * D_KEY
        qh = q_ref[:, lo:lo + D_KEY]
        kh = k_ref[:, lo:lo + D_KEY]
        vh = v_ref[:, lo:lo + D_KEY]
        s = lax.dot_general(qh, kh, _NT, preferred_element_type=jnp.float32)
        s = s * (1.0 / math.sqrt(D_KEY))
        m = jnp.max(s, axis=-1, keepdims=True)
        p = jnp.exp(s - m)
        p = p / jnp.sum(p, axis=-1, keepdims=True)
        outs.append(jnp.dot(p, vh, preferred_element_type=jnp.float32))
    res = jnp.concatenate(outs, axis=1)
    x1 = x_ref[...] + lax.dot_general(res, wo_ref[...], _NT,
                                      preferred_element_type=jnp.float32)
    x1_ref[...] = x1
    f = _rms(x1, sc_ref[...])
    f_ref[...] = f.astype(jnp.bfloat16)
    logits = lax.dot_general(f, gw_ref[...], _NT,
                             preferred_element_type=jnp.float32)
    m = jnp.max(logits, axis=-1, keepdims=True)
    ex = jnp.exp(logits - m)
    probs = ex / jnp.sum(ex, axis=-1, keepdims=True)
    probs_ref[...] = probs
    lane = lax.broadcasted_iota(jnp.int32, probs.shape, 1)
    p0 = jnp.max(probs, axis=-1, keepdims=True)
    e0 = jnp.argmax(probs, axis=-1, keepdims=True).astype(jnp.int32)
    probs2 = jnp.where(lane == e0, -1.0, probs)
    p1 = jnp.max(probs2, axis=-1, keepdims=True)
    e1 = jnp.argmax(probs2, axis=-1, keepdims=True).astype(jnp.int32)
    tot = p0 + p1
    e0_ref[...] = e0
    e1_ref[...] = e1
    g0_ref[...] = p0 / tot
    g1_ref[...] = p1 / tot


def _route_kernel(e0_ref, e1_ref, d0_ref, d1_ref, nb_ref, sb_ref):
    e0 = e0_ref[...]
    e1 = e1_ref[...]
    lane = lax.broadcasted_iota(jnp.int32, (S, E), 1)
    oh0 = (lane == e0).astype(jnp.bfloat16)
    oh1 = (lane == e1).astype(jnp.bfloat16)
    a = oh0 + oh1
    r = lax.broadcasted_iota(jnp.int32, (S, S), 0)
    c = lax.broadcasted_iota(jnp.int32, (S, S), 1)
    tri = (c < r).astype(jnp.bfloat16)
    excl = jnp.dot(tri, a, preferred_element_type=jnp.float32)
    counts = jnp.sum(a.astype(jnp.float32), axis=0, keepdims=True)
    nb = jnp.floor((counts + (BLK - 1)) * (1.0 / BLK))
    tri8 = (lax.broadcasted_iota(jnp.int32, (E, E), 0)
            < lax.broadcasted_iota(jnp.int32, (E, E), 1)).astype(jnp.bfloat16)
    segb = jnp.dot(nb.astype(jnp.bfloat16), tri8,
                   preferred_element_type=jnp.float32) * float(BLK)
    oh0f = oh0.astype(jnp.float32)
    oh1f = oh1.astype(jnp.float32)
    d0 = (jnp.sum(oh0f * segb, axis=1, keepdims=True)
          + jnp.sum(oh0f * excl, axis=1, keepdims=True))
    d1 = (jnp.sum(oh1f * segb, axis=1, keepdims=True)
          + jnp.sum(oh1f * excl, axis=1, keepdims=True))
    d0_ref[...] = d0.astype(jnp.int32)
    d1_ref[...] = d1.astype(jnp.int32)
    nb_ref[...] = nb.astype(jnp.int32)
    sb_ref[...] = segb.astype(jnp.int32)


def _moe_kernel(nb_ref, sb_ref, win_ref, wv_ref, wout_ref, f_ref,
                d0r_ref, d1r_ref, d0c_ref, d1c_ref, g0c_ref, g1c_ref,
                x1_ref, y_ref, fs_ref, eo_ref):
    e = pl.program_id(0)
    c = pl.program_id(1)
    nb = nb_ref[e]
    seg = sb_ref[e]
    inv_sqrt2 = 1.0 / math.sqrt(2.0)

    @pl.when(jnp.logical_and(e == 0, c == 0))
    def _():
        y_ref[...] = x1_ref[...]

    @pl.when(c == 0)
    def _():
        d0r = d0r_ref[...]
        d1r = d1r_ref[...]

        def gather(bi, carry):
            base = seg + bi * BLK
            riota = lax.broadcasted_iota(jnp.int32, (BLK, S), 0) + base
            oh = (riota == d0r) | (riota == d1r)
            ohb = jnp.where(oh, 1.0, 0.0).astype(jnp.bfloat16)
            fs = jnp.dot(ohb, f_ref[...], preferred_element_type=jnp.float32)
            fs_ref[pl.ds(bi * BLK, BLK), :] = fs.astype(jnp.bfloat16)
            return carry

        lax.fori_loop(0, nb, gather, 0)

    win = win_ref[0].astype(jnp.bfloat16)
    wv = wv_ref[0].astype(jnp.bfloat16)
    wout = wout_ref[0].astype(jnp.bfloat16)

    def ffn(bi, carry):
        fsb = fs_ref[pl.ds(bi * BLK, BLK), :]
        h = lax.dot_general(fsb, win, _NT, preferred_element_type=jnp.float32)
        g = 0.5 * h * (1.0 + lax.erf(h * inv_sqrt2))
        v = lax.dot_general(fsb, wv, _NT, preferred_element_type=jnp.float32)
        prod = (g * v).astype(jnp.bfloat16)
        eo = lax.dot_general(prod, wout, _NT,
                             preferred_element_type=jnp.float32)
        sl = pl.ds(bi * BLK, BLK)

        @pl.when(c == 0)
        def _():
            eo_ref[sl, :] = eo

        @pl.when(c != 0)
        def _():
            eo_ref[sl, :] += eo
        return carry

    lax.fori_loop(0, nb, ffn, 0)

    @pl.when(c == NCH - 1)
    def _():
        d0c = d0c_ref[...]
        d1c = d1c_ref[...]
        g0c = g0c_ref[...]
        g1c = g1c_ref[...]

        def scatter(bi, carry):
            base = seg + bi * BLK
            ciota = lax.broadcasted_iota(jnp.int32, (S, BLK), 1) + base
            gt = (jnp.where(ciota == d0c, g0c, 0.0)
                  + jnp.where(ciota == d1c, g1c, 0.0))
            eo = eo_ref[pl.ds(bi * BLK, BLK), :].astype(jnp.bfloat16)
            y_ref[...] += jnp.dot(gt.astype(jnp.bfloat16), eo,
                                  preferred_element_type=jnp.float32)
            return carry

        lax.fori_loop(0, nb, scatter, 0)


@jax.jit
def kernel(x, attn_scale, ffn_scale, Wq, Wk, Wv, Wo, gate_w, We_in, We_v, We_out):
    f32 = jnp.float32
    xs = x.reshape(S, D_MODEL)
    asc = attn_scale.reshape(1, D_MODEL)
    fsc = ffn_scale.reshape(1, D_MODEL)
    wq2 = Wq.reshape(D_MODEL, D_MODEL)
    wk2 = Wk.reshape(D_MODEL, D_MODEL)
    wv2 = Wv.reshape(D_MODEL, D_MODEL)

    nrb = S // SBLK
    full = lambda i: (0, 0)
    rowblk = pl.BlockSpec((SBLK, D_MODEL), lambda i: (i, 0))
    colblk1 = lambda i: pl.BlockSpec((SBLK, 1), lambda j: (j, 0))

    q, k, v = pl.pallas_call(
        _qkv_kernel,
        grid=(nrb,),
        in_specs=[rowblk,
                  pl.BlockSpec((1, D_MODEL), full),
                  pl.BlockSpec((D_MODEL, D_MODEL), full),
                  pl.BlockSpec((D_MODEL, D_MODEL), full),
                  pl.BlockSpec((D_MODEL, D_MODEL), full)],
        out_specs=[rowblk, rowblk, rowblk],
        out_shape=[jax.ShapeDtypeStruct((S, D_MODEL), f32)] * 3,
    )(xs, asc, wq2, wk2, wv2)

    arb = pl.BlockSpec((ABLK, D_MODEL), lambda i: (i, 0))
    x1, fbf, probs, e0, e1, g0, g1 = pl.pallas_call(
        _arr_kernel,
        grid=(S // ABLK,),
        in_specs=[arb, arb,
                  pl.BlockSpec((S, D_MODEL), full),
                  pl.BlockSpec((S, D_MODEL), full),
                  pl.BlockSpec((D_MODEL, D_MODEL), full),
                  pl.BlockSpec((1, D_MODEL), full),
                  pl.BlockSpec((E, D_MODEL), full)],
        out_specs=[arb,
                   pl.BlockSpec((ABLK, D_MODEL), lambda i: (i, 0)),
                   pl.BlockSpec((ABLK, E), lambda i: (i, 0)),
                   pl.BlockSpec((ABLK, 1), lambda i: (i, 0)),
                   pl.BlockSpec((ABLK, 1), lambda i: (i, 0)),
                   pl.BlockSpec((ABLK, 1), lambda i: (i, 0)),
                   pl.BlockSpec((ABLK, 1), lambda i: (i, 0))],
        out_shape=[jax.ShapeDtypeStruct((S, D_MODEL), f32),
                   jax.ShapeDtypeStruct((S, D_MODEL), jnp.bfloat16),
                   jax.ShapeDtypeStruct((S, E), f32),
                   jax.ShapeDtypeStruct((S, 1), jnp.int32),
                   jax.ShapeDtypeStruct((S, 1), jnp.int32),
                   jax.ShapeDtypeStruct((S, 1), f32),
                   jax.ShapeDtypeStruct((S, 1), f32)],
    )(xs, q, k, v, Wo, fsc, gate_w)

    d0, d1, nb, segb = pl.pallas_call(
        _route_kernel,
        grid=(1,),
        in_specs=[pl.BlockSpec((S, 1), full), pl.BlockSpec((S, 1), full)],
        out_specs=[pl.BlockSpec((S, 1), full), pl.BlockSpec((S, 1), full),
                   pl.BlockSpec((1, E), full), pl.BlockSpec((1, E), full)],
        out_shape=[jax.ShapeDtypeStruct((S, 1), jnp.int32),
                   jax.ShapeDtypeStruct((S, 1), jnp.int32),
                   jax.ShapeDtypeStruct((1, E), jnp.int32),
                   jax.ShapeDtypeStruct((1, E), jnp.int32)],
    )(e0, e1)

    d0r = d0.reshape(1, S)
    d1r = d1.reshape(1, S)

    wio = lambda e, c, nbr, sbr: (e, c, 0)
    woo = lambda e, c, nbr, sbr: (e, 0, c)
    cfull = lambda e, c, nbr, sbr: (0, 0)
    y = pl.pallas_call(
        _moe_kernel,
        grid_spec=pltpu.PrefetchScalarGridSpec(
            num_scalar_prefetch=2,
            grid=(E, NCH),
            in_specs=[pl.BlockSpec((1, FCH, D_MODEL), wio),
                      pl.BlockSpec((1, FCH, D_MODEL), wio),
                      pl.BlockSpec((1, D_MODEL, FCH), woo),
                      pl.BlockSpec((S, D_MODEL), cfull),
                      pl.BlockSpec((1, S), cfull),
                      pl.BlockSpec((1, S), cfull),
                      pl.BlockSpec((S, 1), cfull),
                      pl.BlockSpec((S, 1), cfull),
                      pl.BlockSpec((S, 1), cfull),
                      pl.BlockSpec((S, 1), cfull),
                      pl.BlockSpec((S, D_MODEL), cfull)],
            out_specs=pl.BlockSpec((S, D_MODEL), cfull),
            scratch_shapes=[pltpu.VMEM((S, D_MODEL), jnp.bfloat16),
                            pltpu.VMEM((S, D_MODEL), f32)],
        ),
        out_shape=jax.ShapeDtypeStruct((S, D_MODEL), f32),
    )(nb.reshape(E), segb.reshape(E), We_in, We_v, We_out, fbf,
      d0r, d1r, d0, d1, g0, g1, x1)

    return (y.reshape(1, S, D_MODEL), probs.reshape(1, S, E))


# speedup vs baseline: 1.1265x; 1.1265x over previous
import math

import jax
import jax.numpy as jnp
from jax import lax
from jax.experimental import pallas as pl
from jax.experimental.pallas import tpu as pltpu

D_MODEL = 768
HEADS = 12
D_KEY = D_MODEL // HEADS
E = 8
D_FF = D_MODEL * 4
S = 2048
BLK = 256
SBLK = 256
ABLK = 512
NCH = 4
FCH = D_FF // NCH

_NT = (((1,), (1,)), ((), ()))


def _rms(x, scale, eps=1e-5):
    r = jnp.sqrt(jnp.mean(x * x, axis=-1, keepdims=True) + eps)
    return scale * (x / r)


def _qkv_kernel(x_ref, sc_ref, wq_ref, wk_ref, wv_ref, q_ref, k_ref, v_ref):
    a = _rms(x_ref[...], sc_ref[...])
    q_ref[...] = lax.dot_general(a, wq_ref[...], _NT,
                                 preferred_element_type=jnp.float32)
    k_ref[...] = lax.dot_general(a, wk_ref[...], _NT,
                                 preferred_element_type=jnp.float32)
    v_ref[...] = lax.dot_general(a, wv_ref[...], _NT,
                                 preferred_element_type=jnp.float32)


def _arr_kernel(x_ref, q_ref, k_ref, v_ref, wo_ref, sc_ref, gw_ref,
                x1_ref, f_ref, probs_ref, e0_ref, e1_ref, g0_ref, g1_ref):
    outs = []
    for h in range(HEADS):
        lo = h * D_KEY
        qh = q_ref[:, lo:lo + D_KEY]
        kh = k_ref[:, lo:lo + D_KEY]
        vh = v_ref[:, lo:lo + D_KEY]
        s = lax.dot_general(qh, kh, _NT, preferred_element_type=jnp.float32)
        s = s * (1.0 / math.sqrt(D_KEY))
        m = jnp.max(s, axis=-1, keepdims=True)
        p = jnp.exp(s - m)
        p = p / jnp.sum(p, axis=-1, keepdims=True)
        outs.append(jnp.dot(p, vh, preferred_element_type=jnp.float32))
    res = jnp.concatenate(outs, axis=1)
    x1 = x_ref[...] + lax.dot_general(res, wo_ref[...], _NT,
                                      preferred_element_type=jnp.float32)
    x1_ref[...] = x1
    f = _rms(x1, sc_ref[...])
    f_ref[...] = f.astype(jnp.bfloat16)
    logits = lax.dot_general(f, gw_ref[...], _NT,
                             preferred_element_type=jnp.float32)
    m = jnp.max(logits, axis=-1, keepdims=True)
    ex = jnp.exp(logits - m)
    probs = ex / jnp.sum(ex, axis=-1, keepdims=True)
    probs_ref[...] = probs
    lane = lax.broadcasted_iota(jnp.int32, probs.shape, 1)
    p0 = jnp.max(probs, axis=-1, keepdims=True)
    e0 = jnp.argmax(probs, axis=-1, keepdims=True).astype(jnp.int32)
    probs2 = jnp.where(lane == e0, -1.0, probs)
    p1 = jnp.max(probs2, axis=-1, keepdims=True)
    e1 = jnp.argmax(probs2, axis=-1, keepdims=True).astype(jnp.int32)
    tot = p0 + p1
    e0_ref[...] = e0
    e1_ref[...] = e1
    g0_ref[...] = p0 / tot
    g1_ref[...] = p1 / tot


def _route_kernel(e0_ref, e1_ref, d0_ref, d1_ref, nb_ref, sb_ref):
    e0 = e0_ref[...]
    e1 = e1_ref[...]
    lane = lax.broadcasted_iota(jnp.int32, (S, E), 1)
    oh0 = (lane == e0).astype(jnp.bfloat16)
    oh1 = (lane == e1).astype(jnp.bfloat16)
    a = oh0 + oh1
    r = lax.broadcasted_iota(jnp.int32, (S, S), 0)
    c = lax.broadcasted_iota(jnp.int32, (S, S), 1)
    tri = (c < r).astype(jnp.bfloat16)
    excl = jnp.dot(tri, a, preferred_element_type=jnp.float32)
    counts = jnp.sum(a.astype(jnp.float32), axis=0, keepdims=True)
    nb = jnp.floor((counts + (BLK - 1)) * (1.0 / BLK))
    tri8 = (lax.broadcasted_iota(jnp.int32, (E, E), 0)
            < lax.broadcasted_iota(jnp.int32, (E, E), 1)).astype(jnp.bfloat16)
    segb = jnp.dot(nb.astype(jnp.bfloat16), tri8,
                   preferred_element_type=jnp.float32) * float(BLK)
    oh0f = oh0.astype(jnp.float32)
    oh1f = oh1.astype(jnp.float32)
    d0 = (jnp.sum(oh0f * segb, axis=1, keepdims=True)
          + jnp.sum(oh0f * excl, axis=1, keepdims=True))
    d1 = (jnp.sum(oh1f * segb, axis=1, keepdims=True)
          + jnp.sum(oh1f * excl, axis=1, keepdims=True))
    d0_ref[...] = d0.astype(jnp.int32)
    d1_ref[...] = d1.astype(jnp.int32)
    nb_ref[...] = nb.astype(jnp.int32)
    sb_ref[...] = segb.astype(jnp.int32)


def _moe_kernel(nb_ref, sb_ref, win_ref, wv_ref, wout_ref, f_ref,
                d0r_ref, d1r_ref, d0c_ref, d1c_ref, g0c_ref, g1c_ref,
                x1_ref, y_ref, fs_ref, eo_ref):
    e = pl.program_id(0)
    c = pl.program_id(1)
    nb = nb_ref[e]
    seg = sb_ref[e]
    inv_sqrt2 = 1.0 / math.sqrt(2.0)

    @pl.when(jnp.logical_and(e == 0, c == 0))
    def _():
        y_ref[...] = x1_ref[...]

    @pl.when(c == 0)
    def _():
        d0r = d0r_ref[...]
        d1r = d1r_ref[...]

        def gather(bi, carry):
            base = seg + bi * BLK
            riota = lax.broadcasted_iota(jnp.int32, (BLK, S), 0) + base
            oh = (riota == d0r) | (riota == d1r)
            ohb = jnp.where(oh, 1.0, 0.0).astype(jnp.bfloat16)
            fs = jnp.dot(ohb, f_ref[...], preferred_element_type=jnp.float32)
            fs_ref[pl.ds(bi * BLK, BLK), :] = fs.astype(jnp.bfloat16)
            return carry

        lax.fori_loop(0, nb, gather, 0)

    win = win_ref[0].astype(jnp.bfloat16)
    wv = wv_ref[0].astype(jnp.bfloat16)
    wout = wout_ref[0].astype(jnp.bfloat16)

    def ffn(bi, carry):
        fsb = fs_ref[pl.ds(bi * BLK, BLK), :]
        h = lax.dot_general(fsb, win, _NT, preferred_element_type=jnp.float32)
        g = 0.5 * h * (1.0 + lax.erf(h * inv_sqrt2))
        v = lax.dot_general(fsb, wv, _NT, preferred_element_type=jnp.float32)
        prod = (g * v).astype(jnp.bfloat16)
        eo = lax.dot_general(prod, wout, _NT,
                             preferred_element_type=jnp.float32)
        sl = pl.ds(bi * BLK, BLK)

        @pl.when(c == 0)
        def _():
            eo_ref[sl, :] = eo

        @pl.when(c != 0)
        def _():
            eo_ref[sl, :] += eo
        return carry

    lax.fori_loop(0, nb, ffn, 0)

    @pl.when(c == NCH - 1)
    def _():
        d0c = d0c_ref[...]
        d1c = d1c_ref[...]
        g0c = g0c_ref[...]
        g1c = g1c_ref[...]

        def scatter(bi, carry):
            base = seg + bi * BLK
            ciota = lax.broadcasted_iota(jnp.int32, (S, BLK), 1) + base
            gt = (jnp.where(ciota == d0c, g0c, 0.0)
                  + jnp.where(ciota == d1c, g1c, 0.0))
            eo = eo_ref[pl.ds(bi * BLK, BLK), :].astype(jnp.bfloat16)
            y_ref[...] += jnp.dot(gt.astype(jnp.bfloat16), eo,
                                  preferred_element_type=jnp.float32)
            return carry

        lax.fori_loop(0, nb, scatter, 0)


@jax.jit
def kernel(x, attn_scale, ffn_scale, Wq, Wk, Wv, Wo, gate_w, We_in, We_v, We_out):
    f32 = jnp.float32
    xs = x.reshape(S, D_MODEL)
    asc = attn_scale.reshape(1, D_MODEL)
    fsc = ffn_scale.reshape(1, D_MODEL)
    wq2 = Wq.reshape(D_MODEL, D_MODEL)
    wk2 = Wk.reshape(D_MODEL, D_MODEL)
    wv2 = Wv.reshape(D_MODEL, D_MODEL)

    nrb = S // SBLK
    full = lambda i: (0, 0)
    rowblk = pl.BlockSpec((SBLK, D_MODEL), lambda i: (i, 0))
    colblk1 = lambda i: pl.BlockSpec((SBLK, 1), lambda j: (j, 0))

    q, k, v = pl.pallas_call(
        _qkv_kernel,
        grid=(nrb,),
        in_specs=[rowblk,
                  pl.BlockSpec((1, D_MODEL), full),
                  pl.BlockSpec((D_MODEL, D_MODEL), full),
                  pl.BlockSpec((D_MODEL, D_MODEL), full),
                  pl.BlockSpec((D_MODEL, D_MODEL), full)],
        out_specs=[rowblk, rowblk, rowblk],
        out_shape=[jax.ShapeDtypeStruct((S, D_MODEL), f32)] * 3,
    )(xs, asc, wq2, wk2, wv2)

    arb = pl.BlockSpec((ABLK, D_MODEL), lambda i: (i, 0))
    x1, fbf, probs, e0, e1, g0, g1 = pl.pallas_call(
        _arr_kernel,
        grid=(S // ABLK,),
        in_specs=[arb, arb,
                  pl.BlockSpec((S, D_MODEL), full),
                  pl.BlockSpec((S, D_MODEL), full),
                  pl.BlockSpec((D_MODEL, D_MODEL), full),
                  pl.BlockSpec((1, D_MODEL), full),
                  pl.BlockSpec((E, D_MODEL), full)],
        out_specs=[arb,
                   pl.BlockSpec((ABLK, D_MODEL), lambda i: (i, 0)),
                   pl.BlockSpec((ABLK, E), lambda i: (i, 0)),
                   pl.BlockSpec((ABLK, 1), lambda i: (i, 0)),
                   pl.BlockSpec((ABLK, 1), lambda i: (i, 0)),
                   pl.BlockSpec((ABLK, 1), lambda i: (i, 0)),
                   pl.BlockSpec((ABLK, 1), lambda i: (i, 0))],
        out_shape=[jax.ShapeDtypeStruct((S, D_MODEL), f32),
                   jax.ShapeDtypeStruct((S, D_MODEL), jnp.bfloat16),
                   jax.ShapeDtypeStruct((S, E), f32),
                   jax.ShapeDtypeStruct((S, 1), jnp.int32),
                   jax.ShapeDtypeStruct((S, 1), jnp.int32),
                   jax.ShapeDtypeStruct((S, 1), f32),
                   jax.ShapeDtypeStruct((S, 1), f32)],
    )(xs, q, k, v, Wo, fsc, gate_w)

    d0, d1, nb, segb = pl.pallas_call(
        _route_kernel,
        grid=(1,),
        in_specs=[pl.BlockSpec((S, 1), full), pl.BlockSpec((S, 1), full)],
        out_specs=[pl.BlockSpec((S, 1), full), pl.BlockSpec((S, 1), full),
                   pl.BlockSpec((1, E), full), pl.BlockSpec((1, E), full)],
        out_shape=[jax.ShapeDtypeStruct((S, 1), jnp.int32),
                   jax.ShapeDtypeStruct((S, 1), jnp.int32),
                   jax.ShapeDtypeStruct((1, E), jnp.int32),
                   jax.ShapeDtypeStruct((1, E), jnp.int32)],
    )(e0, e1)

    d0r = d0.reshape(1, S)
    d1r = d1.reshape(1, S)

    wio = lambda e, c, nbr, sbr: (e, c, 0)
    woo = lambda e, c, nbr, sbr: (e, 0, c)
    cfull = lambda e, c, nbr, sbr: (0, 0)
    y = pl.pallas_call(
        _moe_kernel,
        grid_spec=pltpu.PrefetchScalarGridSpec(
            num_scalar_prefetch=2,
            grid=(E, NCH),
            in_specs=[pl.BlockSpec((1, FCH, D_MODEL), wio),
                      pl.BlockSpec((1, FCH, D_MODEL), wio),
                      pl.BlockSpec((1, D_MODEL, FCH), woo),
                      pl.BlockSpec((S, D_MODEL), cfull),
                      pl.BlockSpec((1, S), cfull),
                      pl.BlockSpec((1, S), cfull),
                      pl.BlockSpec((S, 1), cfull),
                      pl.BlockSpec((S, 1), cfull),
                      pl.BlockSpec((S, 1), cfull),
                      pl.BlockSpec((S, 1), cfull),
                      pl.BlockSpec((S, D_MODEL), cfull)],
            out_specs=pl.BlockSpec((S, D_MODEL), cfull),
            scratch_shapes=[pltpu.VMEM((S, D_MODEL), jnp.bfloat16),
                            pltpu.VMEM((S, D_MODEL), f32)],
        ),
        out_shape=jax.ShapeDtypeStruct((S, D_MODEL), f32),
    )(nb.reshape(E), segb.reshape(E), We_in, We_v, We_out, fbf,
      d0r, d1r, d0, d1, g0, g1, x1)

    return (y.reshape(1, S, D_MODEL), probs.reshape(1, S, E))


# speedup vs baseline: 1.1320x; 1.0049x over previous
import math

import jax
import jax.numpy as jnp
from jax import lax
from jax.experimental import pallas as pl
from jax.experimental.pallas import tpu as pltpu

D_MODEL = 768
HEADS = 12
D_KEY = D_MODEL // HEADS
E = 8
D_FF = D_MODEL * 4
S = 2048
BLK = 256
SBLK = 256
ABLK = 512
NCH = 4
FCH = D_FF // NCH

_NT = (((1,), (1,)), ((), ()))


def _rms(x, scale, eps=1e-5):
    r = jnp.sqrt(jnp.mean(x * x, axis=-1, keepdims=True) + eps)
    return scale * (x / r)


def _qkv_kernel(x_ref, sc_ref, wq_ref, wk_ref, wv_ref, q_ref, k_ref, v_ref):
    a = _rms(x_ref[...], sc_ref[...])
    q_ref[...] = lax.dot_general(a, wq_ref[...], _NT,
                                 preferred_element_type=jnp.float32)
    k_ref[...] = lax.dot_general(a, wk_ref[...], _NT,
                                 preferred_element_type=jnp.float32)
    v_ref[...] = lax.dot_general(a, wv_ref[...], _NT,
                                 preferred_element_type=jnp.float32)


def _arr_kernel(x_ref, q_ref, k_ref, v_ref, wo_ref, sc_ref, gw_ref,
                x1_ref, f_ref, probs_ref, e0_ref, e1_ref, g0_ref, g1_ref):
    outs = []
    for h in range(HEADS):
        lo = h * D_KEY
        qh = q_ref[:, lo:lo + D_KEY]
        kh = k_ref[:, lo:lo + D_KEY]
        vh = v_ref[:, lo:lo + D_KEY]
        s = lax.dot_general(qh, kh, _NT, preferred_element_type=jnp.float32)
        s = s * (1.0 / math.sqrt(D_KEY))
        m = jnp.max(s, axis=-1, keepdims=True)
        p = jnp.exp(s - m)
        p = p / jnp.sum(p, axis=-1, keepdims=True)
        outs.append(jnp.dot(p, vh, preferred_element_type=jnp.float32))
    res = jnp.concatenate(outs, axis=1)
    x1 = x_ref[...] + lax.dot_general(res, wo_ref[...], _NT,
                                      preferred_element_type=jnp.float32)
    x1_ref[...] = x1
    f = _rms(x1, sc_ref[...])
    f_ref[...] = f.astype(jnp.bfloat16)
    logits = lax.dot_general(f, gw_ref[...], _NT,
                             preferred_element_type=jnp.float32)
    m = jnp.max(logits, axis=-1, keepdims=True)
    ex = jnp.exp(logits - m)
    probs = ex / jnp.sum(ex, axis=-1, keepdims=True)
    probs_ref[...] = probs
    lane = lax.broadcasted_iota(jnp.int32, probs.shape, 1)
    p0 = jnp.max(probs, axis=-1, keepdims=True)
    e0 = jnp.argmax(probs, axis=-1, keepdims=True).astype(jnp.int32)
    probs2 = jnp.where(lane == e0, -1.0, probs)
    p1 = jnp.max(probs2, axis=-1, keepdims=True)
    e1 = jnp.argmax(probs2, axis=-1, keepdims=True).astype(jnp.int32)
    tot = p0 + p1
    e0_ref[...] = e0
    e1_ref[...] = e1
    g0_ref[...] = p0 / tot
    g1_ref[...] = p1 / tot


def _route_kernel(e0_ref, e1_ref, d0_ref, d1_ref, nb_ref, sb_ref):
    e0 = e0_ref[...]
    e1 = e1_ref[...]
    lane = lax.broadcasted_iota(jnp.int32, (S, E), 1)
    oh0 = (lane == e0).astype(jnp.bfloat16)
    oh1 = (lane == e1).astype(jnp.bfloat16)
    a = oh0 + oh1
    r = lax.broadcasted_iota(jnp.int32, (S, S), 0)
    c = lax.broadcasted_iota(jnp.int32, (S, S), 1)
    tri = (c < r).astype(jnp.bfloat16)
    excl = jnp.dot(tri, a, preferred_element_type=jnp.float32)
    counts = jnp.sum(a.astype(jnp.float32), axis=0, keepdims=True)
    nb = jnp.floor((counts + (BLK - 1)) * (1.0 / BLK))
    tri8 = (lax.broadcasted_iota(jnp.int32, (E, E), 0)
            < lax.broadcasted_iota(jnp.int32, (E, E), 1)).astype(jnp.bfloat16)
    segb = jnp.dot(nb.astype(jnp.bfloat16), tri8,
                   preferred_element_type=jnp.float32) * float(BLK)
    oh0f = oh0.astype(jnp.float32)
    oh1f = oh1.astype(jnp.float32)
    d0 = (jnp.sum(oh0f * segb, axis=1, keepdims=True)
          + jnp.sum(oh0f * excl, axis=1, keepdims=True))
    d1 = (jnp.sum(oh1f * segb, axis=1, keepdims=True)
          + jnp.sum(oh1f * excl, axis=1, keepdims=True))
    d0_ref[...] = d0.astype(jnp.int32)
    d1_ref[...] = d1.astype(jnp.int32)
    nb_ref[...] = nb.astype(jnp.int32)
    sb_ref[...] = segb.astype(jnp.int32)


def _moe_kernel(nb_ref, sb_ref, win_ref, wv_ref, wout_ref, f_ref,
                d0r_ref, d1r_ref, d0c_ref, d1c_ref, g0c_ref, g1c_ref,
                x1_ref, y_ref, fs_ref, eo_ref):
    e = pl.program_id(0)
    c = pl.program_id(1)
    nb = nb_ref[e]
    seg = sb_ref[e]
    inv_sqrt2 = 1.0 / math.sqrt(2.0)

    @pl.when(jnp.logical_and(e == 0, c == 0))
    def _():
        y_ref[...] = x1_ref[...]

    @pl.when(c == 0)
    def _():
        d0r = d0r_ref[...]
        d1r = d1r_ref[...]

        def gather(bi, carry):
            base = seg + bi * BLK
            riota = lax.broadcasted_iota(jnp.int32, (BLK, S), 0) + base
            oh = (riota == d0r) | (riota == d1r)
            ohb = jnp.where(oh, 1.0, 0.0).astype(jnp.bfloat16)
            fs = jnp.dot(ohb, f_ref[...], preferred_element_type=jnp.float32)
            fs_ref[pl.ds(bi * BLK, BLK), :] = fs.astype(jnp.bfloat16)
            return carry

        lax.fori_loop(0, nb, gather, 0)

        @pl.when(nb % 2 == 1)
        def _():
            fs_ref[pl.ds(nb * BLK, BLK), :] = jnp.zeros(
                (BLK, D_MODEL), jnp.bfloat16)

    win = win_ref[0].astype(jnp.bfloat16)
    wv = wv_ref[0].astype(jnp.bfloat16)
    wout = wout_ref[0].astype(jnp.bfloat16)

    nh = lax.div(nb + 1, 2)
    PBLK = 2 * BLK

    def ffn(hi, carry):
        fsb = fs_ref[pl.ds(hi * PBLK, PBLK), :]
        h = lax.dot_general(fsb, win, _NT, preferred_element_type=jnp.float32)
        g = 0.5 * h * (1.0 + lax.erf(h * inv_sqrt2))
        v = lax.dot_general(fsb, wv, _NT, preferred_element_type=jnp.float32)
        prod = (g * v).astype(jnp.bfloat16)
        eo = lax.dot_general(prod, wout, _NT,
                             preferred_element_type=jnp.float32)
        sl = pl.ds(hi * PBLK, PBLK)

        @pl.when(c == 0)
        def _():
            eo_ref[sl, :] = eo

        @pl.when(c != 0)
        def _():
            eo_ref[sl, :] += eo
        return carry

    lax.fori_loop(0, nh, ffn, 0)

    @pl.when(c == NCH - 1)
    def _():
        d0c = d0c_ref[...]
        d1c = d1c_ref[...]
        g0c = g0c_ref[...]
        g1c = g1c_ref[...]

        def scatter(hi, carry):
            base = seg + hi * PBLK
            ciota = lax.broadcasted_iota(jnp.int32, (S, PBLK), 1) + base
            gt = (jnp.where(ciota == d0c, g0c, 0.0)
                  + jnp.where(ciota == d1c, g1c, 0.0))
            eo = eo_ref[pl.ds(hi * PBLK, PBLK), :].astype(jnp.bfloat16)
            y_ref[...] += jnp.dot(gt.astype(jnp.bfloat16), eo,
                                  preferred_element_type=jnp.float32)
            return carry

        lax.fori_loop(0, nh, scatter, 0)


@jax.jit
def kernel(x, attn_scale, ffn_scale, Wq, Wk, Wv, Wo, gate_w, We_in, We_v, We_out):
    f32 = jnp.float32
    xs = x.reshape(S, D_MODEL)
    asc = attn_scale.reshape(1, D_MODEL)
    fsc = ffn_scale.reshape(1, D_MODEL)
    wq2 = Wq.reshape(D_MODEL, D_MODEL)
    wk2 = Wk.reshape(D_MODEL, D_MODEL)
    wv2 = Wv.reshape(D_MODEL, D_MODEL)

    nrb = S // SBLK
    full = lambda i: (0, 0)
    rowblk = pl.BlockSpec((SBLK, D_MODEL), lambda i: (i, 0))
    colblk1 = lambda i: pl.BlockSpec((SBLK, 1), lambda j: (j, 0))

    q, k, v = pl.pallas_call(
        _qkv_kernel,
        grid=(nrb,),
        in_specs=[rowblk,
                  pl.BlockSpec((1, D_MODEL), full),
                  pl.BlockSpec((D_MODEL, D_MODEL), full),
                  pl.BlockSpec((D_MODEL, D_MODEL), full),
                  pl.BlockSpec((D_MODEL, D_MODEL), full)],
        out_specs=[rowblk, rowblk, rowblk],
        out_shape=[jax.ShapeDtypeStruct((S, D_MODEL), f32)] * 3,
    )(xs, asc, wq2, wk2, wv2)

    arb = pl.BlockSpec((ABLK, D_MODEL), lambda i: (i, 0))
    x1, fbf, probs, e0, e1, g0, g1 = pl.pallas_call(
        _arr_kernel,
        grid=(S // ABLK,),
        in_specs=[arb, arb,
                  pl.BlockSpec((S, D_MODEL), full),
                  pl.BlockSpec((S, D_MODEL), full),
                  pl.BlockSpec((D_MODEL, D_MODEL), full),
                  pl.BlockSpec((1, D_MODEL), full),
                  pl.BlockSpec((E, D_MODEL), full)],
        out_specs=[arb,
                   pl.BlockSpec((ABLK, D_MODEL), lambda i: (i, 0)),
                   pl.BlockSpec((ABLK, E), lambda i: (i, 0)),
                   pl.BlockSpec((ABLK, 1), lambda i: (i, 0)),
                   pl.BlockSpec((ABLK, 1), lambda i: (i, 0)),
                   pl.BlockSpec((ABLK, 1), lambda i: (i, 0)),
                   pl.BlockSpec((ABLK, 1), lambda i: (i, 0))],
        out_shape=[jax.ShapeDtypeStruct((S, D_MODEL), f32),
                   jax.ShapeDtypeStruct((S, D_MODEL), jnp.bfloat16),
                   jax.ShapeDtypeStruct((S, E), f32),
                   jax.ShapeDtypeStruct((S, 1), jnp.int32),
                   jax.ShapeDtypeStruct((S, 1), jnp.int32),
                   jax.ShapeDtypeStruct((S, 1), f32),
                   jax.ShapeDtypeStruct((S, 1), f32)],
    )(xs, q, k, v, Wo, fsc, gate_w)

    d0, d1, nb, segb = pl.pallas_call(
        _route_kernel,
        grid=(1,),
        in_specs=[pl.BlockSpec((S, 1), full), pl.BlockSpec((S, 1), full)],
        out_specs=[pl.BlockSpec((S, 1), full), pl.BlockSpec((S, 1), full),
                   pl.BlockSpec((1, E), full), pl.BlockSpec((1, E), full)],
        out_shape=[jax.ShapeDtypeStruct((S, 1), jnp.int32),
                   jax.ShapeDtypeStruct((S, 1), jnp.int32),
                   jax.ShapeDtypeStruct((1, E), jnp.int32),
                   jax.ShapeDtypeStruct((1, E), jnp.int32)],
    )(e0, e1)

    d0r = d0.reshape(1, S)
    d1r = d1.reshape(1, S)

    wio = lambda e, c, nbr, sbr: (e, c, 0)
    woo = lambda e, c, nbr, sbr: (e, 0, c)
    cfull = lambda e, c, nbr, sbr: (0, 0)
    y = pl.pallas_call(
        _moe_kernel,
        grid_spec=pltpu.PrefetchScalarGridSpec(
            num_scalar_prefetch=2,
            grid=(E, NCH),
            in_specs=[pl.BlockSpec((1, FCH, D_MODEL), wio),
                      pl.BlockSpec((1, FCH, D_MODEL), wio),
                      pl.BlockSpec((1, D_MODEL, FCH), woo),
                      pl.BlockSpec((S, D_MODEL), cfull),
                      pl.BlockSpec((1, S), cfull),
                      pl.BlockSpec((1, S), cfull),
                      pl.BlockSpec((S, 1), cfull),
                      pl.BlockSpec((S, 1), cfull),
                      pl.BlockSpec((S, 1), cfull),
                      pl.BlockSpec((S, 1), cfull),
                      pl.BlockSpec((S, D_MODEL), cfull)],
            out_specs=pl.BlockSpec((S, D_MODEL), cfull),
            scratch_shapes=[pltpu.VMEM((S, D_MODEL), jnp.bfloat16),
                            pltpu.VMEM((S, D_MODEL), f32)],
        ),
        out_shape=jax.ShapeDtypeStruct((S, D_MODEL), f32),
    )(nb.reshape(E), segb.reshape(E), We_in, We_v, We_out, fbf,
      d0r, d1r, d0, d1, g0, g1, x1)

    return (y.reshape(1, S, D_MODEL), probs.reshape(1, S, E))


# speedup vs baseline: 1.1336x; 1.0014x over previous
import math

import jax
import jax.numpy as jnp
from jax import lax
from jax.experimental import pallas as pl
from jax.experimental.pallas import tpu as pltpu

D_MODEL = 768
HEADS = 12
D_KEY = D_MODEL // HEADS
E = 8
D_FF = D_MODEL * 4
S = 2048
BLK = 256
SBLK = 256
ABLK = 512
NCH = 4
FCH = D_FF // NCH

_NT = (((1,), (1,)), ((), ()))


def _rms(x, scale, eps=1e-5):
    r = jnp.sqrt(jnp.mean(x * x, axis=-1, keepdims=True) + eps)
    return scale * (x / r)


def _qkv_kernel(x_ref, sc_ref, wq_ref, wk_ref, wv_ref, q_ref, k_ref, v_ref):
    a = _rms(x_ref[...], sc_ref[...])
    q_ref[...] = lax.dot_general(a, wq_ref[...], _NT,
                                 preferred_element_type=jnp.float32)
    k_ref[...] = lax.dot_general(a, wk_ref[...], _NT,
                                 preferred_element_type=jnp.float32)
    v_ref[...] = lax.dot_general(a, wv_ref[...], _NT,
                                 preferred_element_type=jnp.float32)


def _arr_kernel(x_ref, q_ref, k_ref, v_ref, wo_ref, sc_ref, gw_ref,
                x1_ref, f_ref, probs_ref, e0_ref, e1_ref, g0_ref, g1_ref):
    outs = []
    for h in range(HEADS):
        lo = h * D_KEY
        qh = q_ref[:, lo:lo + D_KEY]
        kh = k_ref[:, lo:lo + D_KEY]
        vh = v_ref[:, lo:lo + D_KEY]
        s = lax.dot_general(qh, kh, _NT, preferred_element_type=jnp.float32)
        s = s * (1.0 / math.sqrt(D_KEY))
        m = jnp.max(s, axis=-1, keepdims=True)
        p = jnp.exp(s - m)
        p = p / jnp.sum(p, axis=-1, keepdims=True)
        outs.append(jnp.dot(p, vh, preferred_element_type=jnp.float32))
    res = jnp.concatenate(outs, axis=1)
    x1 = x_ref[...] + lax.dot_general(res, wo_ref[...], _NT,
                                      preferred_element_type=jnp.float32)
    x1_ref[...] = x1
    f = _rms(x1, sc_ref[...])
    f_ref[...] = f.astype(jnp.bfloat16)
    logits = lax.dot_general(f, gw_ref[...], _NT,
                             preferred_element_type=jnp.float32)
    m = jnp.max(logits, axis=-1, keepdims=True)
    ex = jnp.exp(logits - m)
    probs = ex / jnp.sum(ex, axis=-1, keepdims=True)
    probs_ref[...] = probs
    lane = lax.broadcasted_iota(jnp.int32, probs.shape, 1)
    p0 = jnp.max(probs, axis=-1, keepdims=True)
    e0 = jnp.argmax(probs, axis=-1, keepdims=True).astype(jnp.int32)
    probs2 = jnp.where(lane == e0, -1.0, probs)
    p1 = jnp.max(probs2, axis=-1, keepdims=True)
    e1 = jnp.argmax(probs2, axis=-1, keepdims=True).astype(jnp.int32)
    tot = p0 + p1
    e0_ref[...] = e0
    e1_ref[...] = e1
    g0_ref[...] = p0 / tot
    g1_ref[...] = p1 / tot


def _route_kernel(e0_ref, e1_ref, d0_ref, d1_ref, nb_ref, sb_ref):
    e0 = e0_ref[...]
    e1 = e1_ref[...]
    lane = lax.broadcasted_iota(jnp.int32, (S, E), 1)
    oh0 = (lane == e0).astype(jnp.bfloat16)
    oh1 = (lane == e1).astype(jnp.bfloat16)
    a = oh0 + oh1
    r = lax.broadcasted_iota(jnp.int32, (S, S), 0)
    c = lax.broadcasted_iota(jnp.int32, (S, S), 1)
    tri = (c < r).astype(jnp.bfloat16)
    excl = jnp.dot(tri, a, preferred_element_type=jnp.float32)
    counts = jnp.sum(a.astype(jnp.float32), axis=0, keepdims=True)
    nb = jnp.floor((counts + (BLK - 1)) * (1.0 / BLK))
    tri8 = (lax.broadcasted_iota(jnp.int32, (E, E), 0)
            < lax.broadcasted_iota(jnp.int32, (E, E), 1)).astype(jnp.bfloat16)
    segb = jnp.dot(nb.astype(jnp.bfloat16), tri8,
                   preferred_element_type=jnp.float32) * float(BLK)
    oh0f = oh0.astype(jnp.float32)
    oh1f = oh1.astype(jnp.float32)
    d0 = (jnp.sum(oh0f * segb, axis=1, keepdims=True)
          + jnp.sum(oh0f * excl, axis=1, keepdims=True))
    d1 = (jnp.sum(oh1f * segb, axis=1, keepdims=True)
          + jnp.sum(oh1f * excl, axis=1, keepdims=True))
    d0_ref[...] = d0.astype(jnp.int32)
    d1_ref[...] = d1.astype(jnp.int32)
    nb_ref[...] = nb.astype(jnp.int32)
    sb_ref[...] = segb.astype(jnp.int32)


def _moe_kernel(nb_ref, sb_ref, win_ref, wv_ref, wout_ref, f_ref,
                d0r_ref, d1r_ref, d0c_ref, d1c_ref, g0c_ref, g1c_ref,
                x1_ref, y_ref, fs_ref, eo_ref):
    e = pl.program_id(0)
    c = pl.program_id(1)
    nb = nb_ref[e]
    seg = sb_ref[e]
    inv_sqrt2 = 1.0 / math.sqrt(2.0)

    @pl.when(jnp.logical_and(e == 0, c == 0))
    def _():
        y_ref[...] = x1_ref[...]

    @pl.when(c == 0)
    def _():
        d0r = d0r_ref[...]
        d1r = d1r_ref[...]

        def gather(bi, carry):
            base = seg + bi * BLK
            riota = lax.broadcasted_iota(jnp.int32, (BLK, S), 0) + base
            oh = (riota == d0r) | (riota == d1r)
            ohb = jnp.where(oh, 1.0, 0.0).astype(jnp.bfloat16)
            fs = jnp.dot(ohb, f_ref[...], preferred_element_type=jnp.float32)
            fs_ref[pl.ds(bi * BLK, BLK), :] = fs.astype(jnp.bfloat16)
            return carry

        lax.fori_loop(0, nb, gather, 0)

        @pl.when(nb % 2 == 1)
        def _():
            fs_ref[pl.ds(nb * BLK, BLK), :] = jnp.zeros(
                (BLK, D_MODEL), jnp.bfloat16)

    win = win_ref[0].astype(jnp.bfloat16)
    wv = wv_ref[0].astype(jnp.bfloat16)
    wout = wout_ref[0].astype(jnp.bfloat16)

    nh = lax.div(nb + 1, 2)
    PBLK = 2 * BLK

    def ffn(hi, carry):
        fsb = fs_ref[pl.ds(hi * PBLK, PBLK), :]
        h = lax.dot_general(fsb, win, _NT, preferred_element_type=jnp.float32)
        g = 0.5 * h * (1.0 + lax.erf(h * inv_sqrt2))
        v = lax.dot_general(fsb, wv, _NT, preferred_element_type=jnp.float32)
        prod = (g * v).astype(jnp.bfloat16)
        eo = lax.dot_general(prod, wout, _NT,
                             preferred_element_type=jnp.float32)
        sl = pl.ds(hi * PBLK, PBLK)

        @pl.when(c == 0)
        def _():
            eo_ref[sl, :] = eo

        @pl.when(c != 0)
        def _():
            eo_ref[sl, :] += eo
        return carry

    lax.fori_loop(0, nh, ffn, 0)

    @pl.when(c == NCH - 1)
    def _():
        d0c = d0c_ref[...]
        d1c = d1c_ref[...]
        g0c = g0c_ref[...]
        g1c = g1c_ref[...]

        def scatter(hi, carry):
            base = seg + hi * PBLK
            ciota = lax.broadcasted_iota(jnp.int32, (S, PBLK), 1) + base
            gt = (jnp.where(ciota == d0c, g0c, 0.0)
                  + jnp.where(ciota == d1c, g1c, 0.0))
            eo = eo_ref[pl.ds(hi * PBLK, PBLK), :].astype(jnp.bfloat16)
            y_ref[...] += jnp.dot(gt.astype(jnp.bfloat16), eo,
                                  preferred_element_type=jnp.float32)
            return carry

        lax.fori_loop(0, nh, scatter, 0)


@jax.jit
def kernel(x, attn_scale, ffn_scale, Wq, Wk, Wv, Wo, gate_w, We_in, We_v, We_out):
    f32 = jnp.float32
    xs = x.reshape(S, D_MODEL)
    asc = attn_scale.reshape(1, D_MODEL)
    fsc = ffn_scale.reshape(1, D_MODEL)
    wq2 = Wq.reshape(D_MODEL, D_MODEL)
    wk2 = Wk.reshape(D_MODEL, D_MODEL)
    wv2 = Wv.reshape(D_MODEL, D_MODEL)

    nrb = S // SBLK
    full = lambda i: (0, 0)
    rowblk = pl.BlockSpec((SBLK, D_MODEL), lambda i: (i, 0))

    q, k, v = pl.pallas_call(
        _qkv_kernel,
        grid=(nrb,),
        in_specs=[rowblk,
                  pl.BlockSpec((1, D_MODEL), full),
                  pl.BlockSpec((D_MODEL, D_MODEL), full),
                  pl.BlockSpec((D_MODEL, D_MODEL), full),
                  pl.BlockSpec((D_MODEL, D_MODEL), full)],
        out_specs=[rowblk, rowblk, rowblk],
        out_shape=[jax.ShapeDtypeStruct((S, D_MODEL), f32)] * 3,
    )(xs, asc, wq2, wk2, wv2)

    arb = pl.BlockSpec((ABLK, D_MODEL), lambda i: (i, 0))
    x1, fbf, probs, e0, e1, g0, g1 = pl.pallas_call(
        _arr_kernel,
        grid=(S // ABLK,),
        in_specs=[arb, arb,
                  pl.BlockSpec((S, D_MODEL), full),
                  pl.BlockSpec((S, D_MODEL), full),
                  pl.BlockSpec((D_MODEL, D_MODEL), full),
                  pl.BlockSpec((1, D_MODEL), full),
                  pl.BlockSpec((E, D_MODEL), full)],
        out_specs=[arb,
                   pl.BlockSpec((ABLK, D_MODEL), lambda i: (i, 0)),
                   pl.BlockSpec((ABLK, E), lambda i: (i, 0)),
                   pl.BlockSpec((ABLK, 1), lambda i: (i, 0)),
                   pl.BlockSpec((ABLK, 1), lambda i: (i, 0)),
                   pl.BlockSpec((ABLK, 1), lambda i: (i, 0)),
                   pl.BlockSpec((ABLK, 1), lambda i: (i, 0))],
        out_shape=[jax.ShapeDtypeStruct((S, D_MODEL), f32),
                   jax.ShapeDtypeStruct((S, D_MODEL), jnp.bfloat16),
                   jax.ShapeDtypeStruct((S, E), f32),
                   jax.ShapeDtypeStruct((S, 1), jnp.int32),
                   jax.ShapeDtypeStruct((S, 1), jnp.int32),
                   jax.ShapeDtypeStruct((S, 1), f32),
                   jax.ShapeDtypeStruct((S, 1), f32)],
    )(xs, q, k, v, Wo, fsc, gate_w)

    d0, d1, nb, segb = pl.pallas_call(
        _route_kernel,
        grid=(1,),
        in_specs=[pl.BlockSpec((S, 1), full), pl.BlockSpec((S, 1), full)],
        out_specs=[pl.BlockSpec((S, 1), full), pl.BlockSpec((S, 1), full),
                   pl.BlockSpec((1, E), full), pl.BlockSpec((1, E), full)],
        out_shape=[jax.ShapeDtypeStruct((S, 1), jnp.int32),
                   jax.ShapeDtypeStruct((S, 1), jnp.int32),
                   jax.ShapeDtypeStruct((1, E), jnp.int32),
                   jax.ShapeDtypeStruct((1, E), jnp.int32)],
    )(e0, e1)

    d0r = d0.reshape(1, S)
    d1r = d1.reshape(1, S)

    wio = lambda e, c, nbr, sbr: (e, c, 0)
    woo = lambda e, c, nbr, sbr: (e, 0, c)
    cfull = lambda e, c, nbr, sbr: (0, 0)
    y = pl.pallas_call(
        _moe_kernel,
        grid_spec=pltpu.PrefetchScalarGridSpec(
            num_scalar_prefetch=2,
            grid=(E, NCH),
            in_specs=[pl.BlockSpec((1, FCH, D_MODEL), wio),
                      pl.BlockSpec((1, FCH, D_MODEL), wio),
                      pl.BlockSpec((1, D_MODEL, FCH), woo),
                      pl.BlockSpec((S, D_MODEL), cfull),
                      pl.BlockSpec((1, S), cfull),
                      pl.BlockSpec((1, S), cfull),
                      pl.BlockSpec((S, 1), cfull),
                      pl.BlockSpec((S, 1), cfull),
                      pl.BlockSpec((S, 1), cfull),
                      pl.BlockSpec((S, 1), cfull),
                      pl.BlockSpec((S, D_MODEL), cfull)],
            out_specs=pl.BlockSpec((S, D_MODEL), cfull),
            scratch_shapes=[pltpu.VMEM((S, D_MODEL), jnp.bfloat16),
                            pltpu.VMEM((S, D_MODEL), f32)],
        ),
        out_shape=jax.ShapeDtypeStruct((S, D_MODEL), f32),
    )(nb.reshape(E), segb.reshape(E), We_in, We_v, We_out, fbf,
      d0r, d1r, d0, d1, g0, g1, x1)

    return (y.reshape(1, S, D_MODEL), probs.reshape(1, S, E))
